# baseline (speedup 1.0000x reference)
import functools

import jax
import jax.numpy as jnp
from jax import lax
from jax.experimental import pallas as pl
from jax.experimental.pallas import tpu as pltpu
from jax.experimental.pallas import tpu_sc as plsc

_NC = 2
_NS = 16
_NW = _NC * _NS
_SUB = 128
_NSUB = 4
_CHUNK = _SUB * _NSUB


def _sc_embedding_lookup(flat_ids_2d, table, n):
    emb = table.shape[1]
    per_w = n // _NW
    chunks_per_w = per_w // _CHUNK
    rows_per_w = per_w // _SUB

    mesh = plsc.VectorSubcoreMesh(core_axis_name="c", subcore_axis_name="s")

    @functools.partial(
        pl.kernel,
        mesh=mesh,
        out_type=jax.ShapeDtypeStruct((n, emb), jnp.float32),
        scratch_types=[
            pltpu.VMEM((_NSUB, _SUB), jnp.int32),
            pltpu.VMEM((_CHUNK, emb), jnp.float32),
            pltpu.SemaphoreType.DMA,
        ],
        compiler_params=pltpu.CompilerParams(use_tc_tiling_on_sc=False),
    )
    def body(ids_hbm, table_hbm, out_hbm, idx_v, rows_v, sem):
        wid = lax.axis_index("s") * _NC + lax.axis_index("c")
        row_base = wid * rows_per_w
        out_base = wid * per_w

        def chunk_body(g, carry):
            pltpu.sync_copy(
                ids_hbm.at[pl.ds(row_base + g * _NSUB, _NSUB)], idx_v
            )
            copies = []
            for j in range(_NSUB):
                copies.append(
                    pltpu.async_copy(
                        table_hbm.at[idx_v.at[j]],
                        rows_v.at[pl.ds(j * _SUB, _SUB)],
                        sem,
                    )
                )
            for c in copies:
                c.wait()
            pltpu.sync_copy(
                rows_v, out_hbm.at[pl.ds(out_base + g * _CHUNK, _CHUNK)]
            )
            return carry

        lax.fori_loop(0, chunks_per_w, chunk_body, 0)

    return body(flat_ids_2d, table)


def kernel(token_ids, table):
    b, s = token_ids.shape
    n = b * s
    flat_ids_2d = token_ids.reshape(n // _SUB, _SUB)
    out = _sc_embedding_lookup(flat_ids_2d, table, n)
    return out.reshape(b, s, table.shape[1])


# speedup vs baseline: 1.0447x; 1.0447x over previous
import functools

import jax
import jax.numpy as jnp
from jax import lax
from jax.experimental import pallas as pl
from jax.experimental.pallas import tpu as pltpu
from jax.experimental.pallas import tpu_sc as plsc

_NC = 2
_NS = 16
_NW = _NC * _NS
_SUB = 128
_NSUB = 4
_CHUNK = _SUB * _NSUB


def _sc_embedding_lookup(flat_ids_2d, table, n):
    emb = table.shape[1]
    per_w = n // _NW
    chunks_per_w = per_w // _CHUNK
    idx_rows_per_w = per_w // _SUB
    assert chunks_per_w % 2 == 0 and chunks_per_w >= 4

    mesh = plsc.VectorSubcoreMesh(core_axis_name="c", subcore_axis_name="s")

    @functools.partial(
        pl.kernel,
        mesh=mesh,
        out_type=jax.ShapeDtypeStruct((n, emb), jnp.float32),
        scratch_types=[
            pltpu.VMEM((idx_rows_per_w, _SUB), jnp.int32),
            pltpu.VMEM((2, _CHUNK, emb), jnp.float32),
            pltpu.SemaphoreType.DMA,
            pltpu.SemaphoreType.DMA,
            pltpu.SemaphoreType.DMA,
            pltpu.SemaphoreType.DMA,
        ],
        compiler_params=pltpu.CompilerParams(use_tc_tiling_on_sc=False),
    )
    def body(ids_hbm, table_hbm, out_hbm, idx_v, rows_v, g0, g1, s0, s1):
        wid = lax.axis_index("s") * _NC + lax.axis_index("c")
        out_base = wid * per_w

        gsems = (g0, g1)
        ssems = (s0, s1)

        def start_gathers(g, b):
            for j in range(_NSUB):
                pltpu.async_copy(
                    table_hbm.at[idx_v.at[g * _NSUB + j]],
                    rows_v.at[b, pl.ds(j * _SUB, _SUB)],
                    gsems[b],
                )

        def wait_gathers(b):
            pltpu.make_async_copy(
                table_hbm.at[pl.ds(0, _CHUNK)], rows_v.at[b], gsems[b]
            ).wait()

        def start_store(g, b):
            pltpu.async_copy(
                rows_v.at[b],
                out_hbm.at[pl.ds(out_base + g * _CHUNK, _CHUNK)],
                ssems[b],
            )

        def wait_store(b):
            pltpu.make_async_copy(
                rows_v.at[b], out_hbm.at[pl.ds(0, _CHUNK)], ssems[b]
            ).wait()

        pltpu.sync_copy(
            ids_hbm.at[pl.ds(wid * idx_rows_per_w, idx_rows_per_w)], idx_v
        )

        start_gathers(0, 0)
        start_gathers(1, 1)
        wait_gathers(0)
        start_store(0, 0)

        def steady(p, carry):
            g = 2 * p + 1
            wait_store(0)
            start_gathers(g + 1, 0)
            wait_gathers(1)
            start_store(g, 1)

            wait_store(1)
            start_gathers(g + 2, 1)
            wait_gathers(0)
            start_store(g + 1, 0)
            return carry

        lax.fori_loop(0, (chunks_per_w - 2) // 2, steady, 0)

        wait_gathers(1)
        start_store(chunks_per_w - 1, 1)
        wait_store(0)
        wait_store(1)

    return body(flat_ids_2d, table)


def kernel(token_ids, table):
    b, s = token_ids.shape
    n = b * s
    flat_ids_2d = token_ids.reshape(n // _SUB, _SUB)
    out = _sc_embedding_lookup(flat_ids_2d, table, n)
    return out.reshape(b, s, table.shape[1])
